# Initial kernel scaffold; baseline (speedup 1.0000x reference)
import jax
import jax.numpy as jnp
from jax.experimental import pallas as pl


def kernel(R, Z, emb, alpha, W_rii, W_rij, W1, W2, Wp):
    raise NotImplementedError("write your pallas kernel here")


# speedup vs baseline: 199.5794x; 199.5794x over previous
import math

import jax
import jax.numpy as jnp
import numpy as np
from jax import lax
from jax.experimental import pallas as pl
from jax.experimental.pallas import tpu as pltpu

_N = 96
_F = 32
_K = 32
_CUTOFF = 15.0
_NP = _N * _N
_E = _N * (_N - 1)

_LOGBINOM = np.asarray(
    [
        math.lgamma(float(_K)) - math.lgamma(k + 1.0) - math.lgamma(float(_K) - k)
        for k in range(_K)
    ],
    dtype=np.float32,
).reshape(1, _K)


def _tc_body(R_ref, Z_ref, emb_ref, alpha_ref, Wrii_ref, Wrij_ref,
             W1_ref, W2_ref, Wp_ref, lb_ref, out_ref):
    f32 = jnp.float32
    R = R_ref[...]
    Ri = jnp.broadcast_to(R[:, None, :], (_N, _N, 3)).reshape(_NP, 3)
    Rj = jnp.broadcast_to(R[None, :, :], (_N, _N, 3)).reshape(_NP, 3)
    diff = Ri - Rj
    d2 = jnp.sum(diff * diff, axis=1, keepdims=True)
    d = jnp.sqrt(d2 + 1e-12)

    alpha = alpha_ref[...]
    sa = jnp.log1p(jnp.exp(alpha))
    ex = jnp.clip(jnp.exp(-sa * d), 1e-10, 1.0 - 1e-10)
    logex = jnp.log(ex)
    log1mex = jnp.log(1.0 - ex)

    kvec = lax.broadcasted_iota(jnp.int32, (1, _K), 1).astype(f32)
    A = lb_ref[...] + kvec * logex + (float(_K) - 1.0 - kvec) * log1mex
    fcut = jnp.where(
        d < _CUTOFF,
        jnp.exp(-d2 / (_CUTOFF * _CUTOFF - d2 + 1e-9)),
        jnp.zeros_like(d),
    )
    pidx = lax.broadcasted_iota(jnp.int32, (_NP, 1), 0)
    offdiag = (pidx % (_N + 1) != 0).astype(f32)
    basis = jnp.exp(A) * (fcut * offdiag)

    g_ii = jnp.dot(basis, Wrii_ref[...], preferred_element_type=f32)
    g_ij = jnp.dot(basis, Wrij_ref[...], preferred_element_type=f32)

    iota87 = lax.broadcasted_iota(jnp.int32, (_N, emb_ref.shape[0]), 1)
    onehot = (Z_ref[...] == iota87).astype(f32)
    x0 = jnp.dot(onehot, emb_ref[...], preferred_element_type=f32)

    x0t = jnp.broadcast_to(x0[None, :, :], (_N, _N, _F)).reshape(_NP, _F)
    agg = jnp.sum((x0t * g_ij).reshape(_N, _N, _F), axis=1)
    x1 = x0 + agg

    t = jnp.dot(x1, W1_ref[...], preferred_element_type=f32)
    sw = t / (1.0 + jnp.exp(-t))
    x2 = x1 + jnp.dot(sw, W2_ref[...], preferred_element_type=f32)

    x2t = jnp.broadcast_to(x2[None, :, :], (_N, _N, _F)).reshape(_NP, _F)
    tsum = jnp.sum((x2t * g_ii).reshape(_N, _N, _F), axis=1)
    rows = jnp.dot(x2 * tsum, Wp_ref[...], preferred_element_type=f32)

    out_ref[...] = jnp.broadcast_to(
        rows[:, None, :], (_N, _N - 1, _F)
    ).reshape(_E, _F)


@jax.jit
def kernel(R, Z, emb, alpha, W_rii, W_rij, W1, W2, Wp):
    Zc = Z.astype(jnp.int32).reshape(_N, 1)
    a = jnp.asarray(alpha, jnp.float32).reshape(1, 1)
    lb = jnp.asarray(_LOGBINOM)
    return pl.pallas_call(
        _tc_body,
        out_shape=jax.ShapeDtypeStruct((_E, _F), jnp.float32),
    )(R, Zc, emb, a, W_rii, W_rij, W1, W2, Wp, lb)
